# Initial kernel scaffold; baseline (speedup 1.0000x reference)
import jax
import jax.numpy as jnp
from jax.experimental import pallas as pl


def kernel(hidden_states, router_w, w_gate, w_up, w_down):
    raise NotImplementedError("write your pallas kernel here")


# speedup vs baseline: 1.4355x; 1.4355x over previous
import functools

import jax
import jax.numpy as jnp
from jax.experimental import pallas as pl
from jax.experimental.pallas import tpu as pltpu

E = 8
D = 768
DFF = 768
TM = 512


def _moe_kernel(x_ref, rw_ref, wg_ref, wu_ref, wd_ref, out_ref):
    e = pl.program_id(0)
    t = pl.program_id(1)
    rows = pl.ds(t * TM, TM)
    x = x_ref[rows, :]

    logits = jax.lax.dot_general(
        x, rw_ref[...], (((1,), (1,)), ((), ())),
        preferred_element_type=jnp.float32)
    lane = jax.lax.broadcasted_iota(jnp.int32, logits.shape, 1)
    e1 = jnp.argmax(logits, axis=-1)[:, None]
    l1 = jnp.max(logits, axis=-1)[:, None]
    masked = jnp.where(lane == e1, -jnp.inf, logits)
    e2 = jnp.argmax(masked, axis=-1)[:, None]
    l2 = jnp.max(masked, axis=-1)[:, None]
    z = jnp.exp(l2 - l1)
    w1 = 1.0 / (1.0 + z)
    w2 = 1.0 - w1
    cw = jnp.where(e1 == e, w1, 0.0) + jnp.where(e2 == e, w2, 0.0)

    wg = wg_ref[0]
    wu = wu_ref[0]
    wd = wd_ref[0]
    g = jax.lax.dot_general(x, wg, (((1,), (1,)), ((), ())),
                            preferred_element_type=jnp.float32)
    u = jax.lax.dot_general(x, wu, (((1,), (1,)), ((), ())),
                            preferred_element_type=jnp.float32)
    h = g * jax.nn.sigmoid(g) * u
    o = jax.lax.dot_general(h, wd, (((1,), (1,)), ((), ())),
                            preferred_element_type=jnp.float32)
    contrib = o * cw

    @pl.when(e == 0)
    def _init():
        out_ref[rows, :] = contrib

    @pl.when(e != 0)
    def _acc():
        out_ref[rows, :] += contrib


@functools.partial(jax.jit, static_argnames=())
def kernel(hidden_states, router_w, w_gate, w_up, w_down):
    orig_shape = hidden_states.shape
    x = hidden_states.reshape(-1, orig_shape[-1])
    T = x.shape[0]
    grid = (E, T // TM)
    y = pl.pallas_call(
        _moe_kernel,
        grid=grid,
        in_specs=[
            pl.BlockSpec((T, D), lambda e, t: (0, 0)),
            pl.BlockSpec((E, D), lambda e, t: (0, 0)),
            pl.BlockSpec((1, DFF, D), lambda e, t: (e, 0, 0)),
            pl.BlockSpec((1, DFF, D), lambda e, t: (e, 0, 0)),
            pl.BlockSpec((1, D, DFF), lambda e, t: (e, 0, 0)),
        ],
        out_specs=pl.BlockSpec((T, D), lambda e, t: (0, 0)),
        out_shape=jax.ShapeDtypeStruct((T, D), jnp.float32),
        compiler_params=pltpu.CompilerParams(
            dimension_semantics=("arbitrary", "arbitrary"),
        ),
    )(x, router_w, w_gate, w_up, w_down)
    return y.reshape(orig_shape)
